# Initial kernel scaffold; baseline (speedup 1.0000x reference)
import jax
import jax.numpy as jnp
from jax.experimental import pallas as pl


def kernel(gx, logits, embed_weight, cur_token_ids):
    raise NotImplementedError("write your pallas kernel here")


# speedup vs baseline: 1.0068x; 1.0068x over previous
import jax
import jax.numpy as jnp
from jax.experimental import pallas as pl

EPS = 1e-10
PROMPT_LENGTH = 32
K_VAL = 50

_VB = 1024


def _bias_body(ce_ref, w_ref, out_ref):
    ce = ce_ref[...]
    w = w_ref[...]
    t1 = jnp.sum(w * w, axis=1)
    t3 = jnp.sum(ce * ce, axis=1)
    t2 = jax.lax.dot_general(
        ce.astype(jnp.bfloat16), w.astype(jnp.bfloat16),
        (((1,), (1,)), ((), ())), preferred_element_type=jnp.float32)
    out_ref[...] = 2.0 * t2 - t1[None, :] - t3[:, None]


def _bias_matmul(ce, embed_weight):
    R, E = ce.shape
    V = embed_weight.shape[0]
    nv = pl.cdiv(V, _VB)
    return pl.pallas_call(
        _bias_body,
        grid=(nv,),
        in_specs=[
            pl.BlockSpec((R, E), lambda j: (0, 0)),
            pl.BlockSpec((_VB, E), lambda j: (j, 0)),
        ],
        out_specs=pl.BlockSpec((R, _VB), lambda j: (0, j)),
        out_shape=jax.ShapeDtypeStruct((R, V), jnp.float32),
    )(ce, embed_weight)


def kernel(gx, logits, embed_weight, cur_token_ids):
    B, L, V = gx.shape
    logits_s = logits[:, PROMPT_LENGTH:, :]
    cur_ids_s = cur_token_ids[:, PROMPT_LENGTH:]

    _, topk_ids = jax.lax.top_k(logits_s, K_VAL)
    dist_logits = jnp.take_along_axis(-gx, topk_ids, axis=-1)
    dist_logits = jnp.where(topk_ids == cur_ids_s[..., None],
                            dist_logits * EPS, dist_logits)
    gumbel = jax.random.gumbel(jax.random.key(42), dist_logits.shape,
                               dtype=dist_logits.dtype)
    sampled = jnp.argmax(dist_logits + gumbel, axis=-1)
    actual_ids = jnp.take_along_axis(topk_ids, sampled[..., None], axis=-1)[..., 0]

    ce = jnp.take(embed_weight, actual_ids.reshape(-1), axis=0)
    bias = _bias_matmul(ce, embed_weight)
    return bias.reshape(B, L, V)


# speedup vs baseline: 5.2119x; 5.1764x over previous
import functools

import jax
import jax.numpy as jnp
from jax import lax
from jax.experimental import pallas as pl
from jax.experimental.pallas import tpu as pltpu
from jax.experimental.pallas import tpu_sc as plsc

EPS = 1e-10
PROMPT_LENGTH = 32
K_VAL = 50

V = 50257
E = 768
VPAD = 50272
NVEC = VPAD // 16
NEG_INF = float("-inf")
IMIN = -2147483648

_VB = 1024

_NC = 2
_NS = 16
_NW = _NC * _NS


def _splat(x):
    return jnp.full((16,), x, jnp.int32)


def _hist_walk(hist, slab_base, krem):
    lanes = lax.iota(jnp.int32, 16)
    zero = _splat(0)

    def chunk(c2, carry):
        carry_cnt, done, dstar, cntab = carry
        c = 15 - c2
        didx = jnp.left_shift(_splat(c * 16) + lanes, 4)
        t = zero
        for l in range(16):
            t = t + plsc.load_gather(hist, [didx + _splat(slab_base + l)])
        r = lax.rev(t, (0,))
        cs = plsc.cumsum(r) + carry_cnt
        m = cs >= krem
        found = jnp.any(m)
        j0 = plsc.all_reduce_ffs(m)
        dstar_new = _splat(c * 16 + 15) - j0
        below = jnp.where(lanes < j0, r, zero)
        cntab_new = carry_cnt + _splat(jnp.sum(below))
        upd = jnp.logical_and(found, jnp.logical_not(done))
        dstar = jnp.where(upd, dstar_new, dstar)
        cntab = jnp.where(upd, cntab_new, cntab)
        done = jnp.logical_or(done, found)
        carry_cnt = carry_cnt + _splat(jnp.sum(t))
        return carry_cnt, done, dstar, cntab

    init = (zero, jnp.zeros((16,), jnp.bool_), zero, zero)
    _, _, dstar, cntab = lax.fori_loop(0, 16, chunk, init)
    return dstar, cntab


_STAGE = "C"


def _sc_body(lg, gx1, cur, gum_h, emb, ce_out, ids_out,
             row_v, hist, cand_v, cand_i, gflat, gxv, gum_v, cur_v,
             ids_v, ce_v, sem):
    wid = lax.axis_index("s") * _NC + lax.axis_index("c")
    base = wid * 16
    lanes = lax.iota(jnp.int32, 16)
    ones = _splat(1)
    zero = _splat(0)
    izeros = jnp.zeros((16,), jnp.int32)
    fzeros = jnp.zeros((16,), jnp.float32)

    pltpu.sync_copy(cur.at[pl.ds(base, 16)], cur_v)

    def row_step(i, aids):
        r = base + i
        pltpu.sync_copy(lg.at[r], row_v)
        pltpu.sync_copy(gum_h.at[r], gum_v)

        if _STAGE == "A":
            return jnp.where(lanes == _splat(i), _splat(r), aids)

        def zr(j, c):
            hist[pl.ds(j * 16, 16)] = izeros
            return c
        lax.fori_loop(0, 1024, zr, 0)
        for a in range(4):
            cand_v[pl.ds(a * 16, 16)] = izeros
            cand_i[pl.ds(a * 16, 16)] = izeros

        def p0(j, c):
            x = row_v[pl.ds(j * 16, 16)]
            b = lax.bitcast_convert_type(x, jnp.int32)
            s = jnp.right_shift(b, 31)
            v = jnp.bitwise_xor(b, jnp.bitwise_and(s, 0x7FFFFFFF))
            row_v[pl.ds(j * 16, 16)] = lax.bitcast_convert_type(v, jnp.float32)
            d1 = jnp.right_shift(v, 24) + 128
            idx = jnp.left_shift(d1, 4) + lanes
            plsc.addupdate_scatter(hist, [idx], ones)
            return c
        lax.fori_loop(0, NVEC, p0, 0)

        krem = _splat(K_VAL)
        b1, cntab = _hist_walk(hist, 0, krem)
        krem = krem - cntab
        p1 = b1 - 128

        def p2s(j, c):
            v = lax.bitcast_convert_type(row_v[pl.ds(j * 16, 16)], jnp.int32)
            pm = jnp.right_shift(v, 24) == p1
            d = jnp.bitwise_and(jnp.right_shift(v, 16), 0xFF)
            idx = 4096 + jnp.left_shift(d, 4) + lanes
            plsc.addupdate_scatter(hist, [idx], ones, mask=pm)
            return c
        lax.fori_loop(0, NVEC, p2s, 0)
        b2, cntab = _hist_walk(hist, 4096, krem)
        krem = krem - cntab
        p2 = p1 * 256 + b2

        def p3s(j, c):
            v = lax.bitcast_convert_type(row_v[pl.ds(j * 16, 16)], jnp.int32)
            pm = jnp.right_shift(v, 16) == p2
            d = jnp.bitwise_and(jnp.right_shift(v, 8), 0xFF)
            idx = 8192 + jnp.left_shift(d, 4) + lanes
            plsc.addupdate_scatter(hist, [idx], ones, mask=pm)
            return c
        lax.fori_loop(0, NVEC, p3s, 0)
        b3, cntab = _hist_walk(hist, 8192, krem)
        krem = krem - cntab
        p3 = p2 * 256 + b3

        def p4s(j, c):
            v = lax.bitcast_convert_type(row_v[pl.ds(j * 16, 16)], jnp.int32)
            pm = jnp.right_shift(v, 8) == p3
            d = jnp.bitwise_and(v, 0xFF)
            idx = 12288 + jnp.left_shift(d, 4) + lanes
            plsc.addupdate_scatter(hist, [idx], ones, mask=pm)
            return c
        lax.fori_loop(0, NVEC, p4s, 0)
        b4, cntab = _hist_walk(hist, 12288, krem)
        quota = krem - cntab
        tval = p3 * 256 + b4

        if _STAGE == "B":
            aid = (jnp.abs(tval) + quota) % _splat(V)
            return jnp.where(lanes == _splat(i), aid, aids)

        def coll(j, carry):
            ncoll, eqcnt = carry
            v = lax.bitcast_convert_type(row_v[pl.ds(j * 16, 16)], jnp.int32)
            gt = v > tval
            eq = v == tval
            pref = plsc.cumsum(eq.astype(jnp.int32)) + eqcnt
            acc = jnp.logical_and(eq, pref <= quota)
            msk = jnp.logical_or(gt, acc)
            pos = plsc.cumsum(msk.astype(jnp.int32)) - 1 + ncoll
            pos = jnp.where(msk, pos, _splat(64) + lanes)
            gidx = _splat(j * 16) + lanes
            plsc.addupdate_scatter(cand_v, [pos], v, mask=msk)
            plsc.addupdate_scatter(cand_i, [pos], gidx, mask=msk)
            ncoll = ncoll + _splat(jnp.sum(msk.astype(jnp.int32)))
            eqcnt = eqcnt + _splat(jnp.sum(eq.astype(jnp.int32)))
            return ncoll, eqcnt
        ncoll_f, _eq_f = lax.fori_loop(0, NVEC, coll, (zero, zero))

        if _STAGE == "C0":
            aid = ncoll_f % _splat(V)
            return jnp.where(lanes == _splat(i), aid, aids)

        rv = r * V
        tail = cand_v[pl.ds(48, 16)]
        cand_v[pl.ds(48, 16)] = jnp.where(lanes < 2, tail, _splat(IMIN))
        cu = []
        ci = []
        for a in range(4):
            cu.append(cand_v[pl.ds(a * 16, 16)])
            ci.append(cand_i[pl.ds(a * 16, 16)])
            safe = jnp.clip(ci[a], 0, V - 1)
            gflat[pl.ds(a * 16, 16)] = safe + _splat(rv)

        if _STAGE == "C1":
            aid = _splat(jnp.max(jnp.maximum(jnp.maximum(ci[0], ci[1]),
                                             jnp.maximum(ci[2], ci[3]))))
            return jnp.where(lanes == _splat(i), aid, aids)
        pltpu.async_copy(gx1.at[gflat], gxv, sem).wait()

        def rank_step(t, rk):
            ts = _splat(t)
            ub = plsc.load_gather(cand_v, [ts])
            ib = plsc.load_gather(cand_i, [ts])
            out = []
            for a in range(4):
                c = jnp.logical_or(
                    ub > cu[a],
                    jnp.logical_and(ub == cu[a], ib < ci[a]))
                out.append(rk[a] + c.astype(jnp.int32))
            return tuple(out)
        ranks = lax.fori_loop(0, 64, rank_step, (izeros, izeros, izeros, izeros))

        curid = plsc.load_gather(cur_v, [_splat(i)])
        best = jnp.full((16,), NEG_INF, jnp.float32)
        scs = []
        for a in range(4):
            gv = plsc.load_gather(gum_v, [ranks[a]])
            xg = gxv[pl.ds(a * 16, 16)]
            neg = -xg
            dist = jnp.where(ci[a] == curid, neg * EPS, neg)
            sc = dist + gv
            if a == 3:
                sc = jnp.where(lanes < 2, sc, jnp.full((16,), NEG_INF, jnp.float32))
            scs.append(sc)
            best = jnp.maximum(best, sc)
        msp = jnp.full((16,), jnp.max(best), jnp.float32)
        rmin = _splat(64)
        for a in range(4):
            rmin = jnp.minimum(rmin, jnp.where(scs[a] == msp, ranks[a], _splat(64)))
        rmin = _splat(jnp.min(rmin))
        aid = zero
        for a in range(4):
            aid = aid + jnp.where(ranks[a] == rmin, ci[a], zero)
        aid = _splat(jnp.sum(aid))
        return jnp.where(lanes == _splat(i), aid, aids)

    aids = lax.fori_loop(0, 16, row_step, izeros)
    ids_v[...] = jnp.clip(aids, 0, V - 1)
    pltpu.async_copy(emb.at[ids_v], ce_v, sem).wait()
    pltpu.sync_copy(ce_v, ce_out.at[pl.ds(base, 16)])
    pltpu.sync_copy(ids_v, ids_out.at[pl.ds(base, 16)])


@jax.jit
def _sc_sample(lg_pad, gx1, cur, gum, emb):
    mesh = plsc.VectorSubcoreMesh(core_axis_name="c", subcore_axis_name="s")
    f = pl.kernel(
        _sc_body,
        out_type=[
            jax.ShapeDtypeStruct((_NW * 16, E), jnp.float32),
            jax.ShapeDtypeStruct((_NW * 16,), jnp.int32),
        ],
        mesh=mesh,
        compiler_params=pltpu.CompilerParams(needs_layout_passes=False),
        scratch_types=[
            pltpu.VMEM((VPAD,), jnp.float32),
            pltpu.VMEM((16384,), jnp.int32),
            pltpu.VMEM((96,), jnp.int32),
            pltpu.VMEM((96,), jnp.int32),
            pltpu.VMEM((64,), jnp.int32),
            pltpu.VMEM((64,), jnp.float32),
            pltpu.VMEM((80,), jnp.float32),
            pltpu.VMEM((16,), jnp.int32),
            pltpu.VMEM((16,), jnp.int32),
            pltpu.VMEM((16, E), jnp.float32),
            pltpu.SemaphoreType.DMA,
        ],
    )
    return f(lg_pad, gx1, cur, gum, emb)


def _bias_body(ce_ref, w_ref, out_ref):
    ce = ce_ref[...]
    w = w_ref[...]
    t1 = jnp.sum(w * w, axis=1)
    t3 = jnp.sum(ce * ce, axis=1)
    t2 = lax.dot_general(
        ce.astype(jnp.bfloat16), w.astype(jnp.bfloat16),
        (((1,), (1,)), ((), ())), preferred_element_type=jnp.float32)
    out_ref[...] = 2.0 * t2 - t1[None, :] - t3[:, None]


def _bias_matmul(ce, embed_weight):
    R = ce.shape[0]
    nv = pl.cdiv(V, _VB)
    return pl.pallas_call(
        _bias_body,
        grid=(nv,),
        in_specs=[
            pl.BlockSpec((R, E), lambda j: (0, 0)),
            pl.BlockSpec((_VB, E), lambda j: (j, 0)),
        ],
        out_specs=pl.BlockSpec((R, _VB), lambda j: (0, j)),
        out_shape=jax.ShapeDtypeStruct((R, V), jnp.float32),
    )(ce, embed_weight)


def kernel(gx, logits, embed_weight, cur_token_ids):
    B, L, _ = gx.shape
    lg_pad = jnp.pad(
        logits[:, PROMPT_LENGTH:, :].reshape(B * L, V),
        ((0, 0), (0, VPAD - V)), constant_values=NEG_INF)
    gx1 = gx.reshape(-1)
    cur = cur_token_ids[:, PROMPT_LENGTH:].reshape(-1)
    gum = jax.random.gumbel(jax.random.key(42), (B, L, K_VAL), dtype=jnp.float32)
    gum = jnp.pad(gum.reshape(B * L, K_VAL), ((0, 0), (0, 80 - K_VAL)))

    ce, _ids = _sc_sample(lg_pad, gx1, cur, gum, embed_weight)
    bias = _bias_matmul(ce, embed_weight)
    return bias.reshape(B, L, V)


# speedup vs baseline: 11.8064x; 2.2653x over previous
import functools

import jax
import jax.numpy as jnp
from jax import lax
from jax.experimental import pallas as pl
from jax.experimental.pallas import tpu as pltpu
from jax.experimental.pallas import tpu_sc as plsc

EPS = 1e-10
PROMPT_LENGTH = 32
K_VAL = 50

V = 50257
E = 768
VPAD = 50272
NVEC = VPAD // 16
NEG_INF = float("-inf")
IMIN = -2147483648

_VB = 1024

_NC = 2
_NS = 16
_NW = _NC * _NS


def _splat(x):
    return jnp.full((16,), x, jnp.int32)


def _hist_walk(hist, slab_base, krem):
    lanes = lax.iota(jnp.int32, 16)
    zero = _splat(0)

    def chunk(c2, carry):
        carry_cnt, done, dstar, cntab = carry
        c = 15 - c2
        didx = jnp.left_shift(_splat(c * 16) + lanes, 4)
        t = zero
        for l in range(16):
            t = t + plsc.load_gather(hist, [didx + _splat(slab_base + l)])
        r = lax.rev(t, (0,))
        cs = plsc.cumsum(r) + carry_cnt
        m = cs >= krem
        found = jnp.any(m)
        j0 = plsc.all_reduce_ffs(m)
        dstar_new = _splat(c * 16 + 15) - j0
        below = jnp.where(lanes < j0, r, zero)
        cntab_new = carry_cnt + _splat(jnp.sum(below))
        upd = jnp.logical_and(found, jnp.logical_not(done))
        dstar = jnp.where(upd, dstar_new, dstar)
        cntab = jnp.where(upd, cntab_new, cntab)
        done = jnp.logical_or(done, found)
        carry_cnt = carry_cnt + _splat(jnp.sum(t))
        return carry_cnt, done, dstar, cntab

    init = (zero, jnp.zeros((16,), jnp.bool_), zero, zero)
    _, _, dstar, cntab = lax.fori_loop(0, 16, chunk, init)
    return dstar, cntab


_STAGE = "C"


def _sc_body(lg, gx1, cur, gum_h, emb, ce_out, ids_out,
             row_v, hist, cand_v, cand_i, gflat, gxv, gum_v, cur_v,
             ids_v, ce_v, sem):
    wid = lax.axis_index("s") * _NC + lax.axis_index("c")
    base = wid * 16
    lanes = lax.iota(jnp.int32, 16)
    ones = _splat(1)
    zero = _splat(0)
    izeros = jnp.zeros((16,), jnp.int32)
    fzeros = jnp.zeros((16,), jnp.float32)

    pltpu.sync_copy(cur.at[pl.ds(base, 16)], cur_v)

    def row_step(i, aids):
        r = base + i
        pltpu.sync_copy(lg.at[r], row_v)
        pltpu.sync_copy(gum_h.at[r], gum_v)

        if _STAGE == "A":
            return jnp.where(lanes == _splat(i), _splat(r), aids)

        @plsc.parallel_loop(0, 1024, unroll=8)
        def _zr(j):
            hist[pl.ds(j * 16, 16)] = izeros
        for a in range(4):
            cand_v[pl.ds(a * 16, 16)] = izeros
            cand_i[pl.ds(a * 16, 16)] = izeros

        @plsc.parallel_loop(0, NVEC, unroll=4)
        def _p0(j):
            x = row_v[pl.ds(j * 16, 16)]
            b = lax.bitcast_convert_type(x, jnp.int32)
            s = jnp.right_shift(b, 31)
            v = jnp.bitwise_xor(b, jnp.bitwise_and(s, 0x7FFFFFFF))
            row_v[pl.ds(j * 16, 16)] = lax.bitcast_convert_type(v, jnp.float32)
            d1 = jnp.right_shift(v, 24) + 128
            idx = jnp.left_shift(d1, 4) + lanes
            plsc.addupdate_scatter(hist, [idx], ones)

        krem = _splat(K_VAL)
        b1, cntab = _hist_walk(hist, 0, krem)
        krem = krem - cntab
        p1 = b1 - 128

        @plsc.parallel_loop(0, NVEC, unroll=4)
        def _p2s(j):
            v = lax.bitcast_convert_type(row_v[pl.ds(j * 16, 16)], jnp.int32)
            pm = jnp.right_shift(v, 24) == p1
            d = jnp.bitwise_and(jnp.right_shift(v, 16), 0xFF)
            idx = 4096 + jnp.left_shift(d, 4) + lanes
            plsc.addupdate_scatter(hist, [idx], ones, mask=pm)
        b2, cntab = _hist_walk(hist, 4096, krem)
        krem = krem - cntab
        p2 = p1 * 256 + b2

        @plsc.parallel_loop(0, NVEC, unroll=4)
        def _p3s(j):
            v = lax.bitcast_convert_type(row_v[pl.ds(j * 16, 16)], jnp.int32)
            pm = jnp.right_shift(v, 16) == p2
            d = jnp.bitwise_and(jnp.right_shift(v, 8), 0xFF)
            idx = 8192 + jnp.left_shift(d, 4) + lanes
            plsc.addupdate_scatter(hist, [idx], ones, mask=pm)
        b3, cntab = _hist_walk(hist, 8192, krem)
        krem = krem - cntab
        p3 = p2 * 256 + b3

        @plsc.parallel_loop(0, NVEC, unroll=4)
        def _p4s(j):
            v = lax.bitcast_convert_type(row_v[pl.ds(j * 16, 16)], jnp.int32)
            pm = jnp.right_shift(v, 8) == p3
            d = jnp.bitwise_and(v, 0xFF)
            idx = 12288 + jnp.left_shift(d, 4) + lanes
            plsc.addupdate_scatter(hist, [idx], ones, mask=pm)
        b4, cntab = _hist_walk(hist, 12288, krem)
        quota = krem - cntab
        tval = p3 * 256 + b4

        if _STAGE == "B":
            aid = (jnp.abs(tval) + quota) % _splat(V)
            return jnp.where(lanes == _splat(i), aid, aids)

        @plsc.parallel_loop(0, NVEC, unroll=2, carry=(zero, zero))
        def coll(j, carry):
            ncoll, eqcnt = carry
            v = lax.bitcast_convert_type(row_v[pl.ds(j * 16, 16)], jnp.int32)
            gt = v > tval
            eq = v == tval
            pref = plsc.cumsum(eq.astype(jnp.int32)) + eqcnt
            acc = jnp.logical_and(eq, pref <= quota)
            msk = jnp.logical_or(gt, acc)
            pos = plsc.cumsum(msk.astype(jnp.int32)) - 1 + ncoll
            pos = jnp.where(msk, pos, _splat(64) + lanes)
            gidx = _splat(j * 16) + lanes
            plsc.addupdate_scatter(cand_v, [pos], v, mask=msk)
            plsc.addupdate_scatter(cand_i, [pos], gidx, mask=msk)
            ncoll = ncoll + _splat(jnp.sum(msk.astype(jnp.int32)))
            eqcnt = eqcnt + _splat(jnp.sum(eq.astype(jnp.int32)))
            return ncoll, eqcnt
        ncoll_f, _eq_f = coll

        if _STAGE == "C0":
            aid = ncoll_f % _splat(V)
            return jnp.where(lanes == _splat(i), aid, aids)

        rv = r * V
        tail = cand_v[pl.ds(48, 16)]
        cand_v[pl.ds(48, 16)] = jnp.where(lanes < 2, tail, _splat(IMIN))
        cu = []
        ci = []
        for a in range(4):
            cu.append(cand_v[pl.ds(a * 16, 16)])
            ci.append(cand_i[pl.ds(a * 16, 16)])
            safe = jnp.clip(ci[a], 0, V - 1)
            gflat[pl.ds(a * 16, 16)] = safe + _splat(rv)

        if _STAGE == "C1":
            aid = _splat(jnp.max(jnp.maximum(jnp.maximum(ci[0], ci[1]),
                                             jnp.maximum(ci[2], ci[3]))))
            return jnp.where(lanes == _splat(i), aid, aids)
        pltpu.async_copy(gx1.at[gflat], gxv, sem).wait()

        @plsc.parallel_loop(0, 64, unroll=4,
                            carry=(izeros, izeros, izeros, izeros))
        def rank_step(t, rk):
            ts = _splat(t)
            ub = plsc.load_gather(cand_v, [ts])
            ib = plsc.load_gather(cand_i, [ts])
            out = []
            for a in range(4):
                c = jnp.logical_or(
                    ub > cu[a],
                    jnp.logical_and(ub == cu[a], ib < ci[a]))
                out.append(rk[a] + c.astype(jnp.int32))
            return tuple(out)
        ranks = rank_step

        curid = plsc.load_gather(cur_v, [_splat(i)])
        best = jnp.full((16,), NEG_INF, jnp.float32)
        scs = []
        for a in range(4):
            gv = plsc.load_gather(gum_v, [ranks[a]])
            xg = gxv[pl.ds(a * 16, 16)]
            neg = -xg
            dist = jnp.where(ci[a] == curid, neg * EPS, neg)
            sc = dist + gv
            if a == 3:
                sc = jnp.where(lanes < 2, sc, jnp.full((16,), NEG_INF, jnp.float32))
            scs.append(sc)
            best = jnp.maximum(best, sc)
        msp = jnp.full((16,), jnp.max(best), jnp.float32)
        rmin = _splat(64)
        for a in range(4):
            rmin = jnp.minimum(rmin, jnp.where(scs[a] == msp, ranks[a], _splat(64)))
        rmin = _splat(jnp.min(rmin))
        aid = zero
        for a in range(4):
            aid = aid + jnp.where(ranks[a] == rmin, ci[a], zero)
        aid = _splat(jnp.sum(aid))
        return jnp.where(lanes == _splat(i), aid, aids)

    aids = lax.fori_loop(0, 16, row_step, izeros)
    ids_v[...] = jnp.clip(aids, 0, V - 1)
    pltpu.async_copy(emb.at[ids_v], ce_v, sem).wait()
    pltpu.sync_copy(ce_v, ce_out.at[pl.ds(base, 16)])
    pltpu.sync_copy(ids_v, ids_out.at[pl.ds(base, 16)])


@jax.jit
def _sc_sample(lg_pad, gx1, cur, gum, emb):
    mesh = plsc.VectorSubcoreMesh(core_axis_name="c", subcore_axis_name="s")
    f = pl.kernel(
        _sc_body,
        out_type=[
            jax.ShapeDtypeStruct((_NW * 16, E), jnp.float32),
            jax.ShapeDtypeStruct((_NW * 16,), jnp.int32),
        ],
        mesh=mesh,
        compiler_params=pltpu.CompilerParams(needs_layout_passes=False),
        scratch_types=[
            pltpu.VMEM((VPAD,), jnp.float32),
            pltpu.VMEM((16384,), jnp.int32),
            pltpu.VMEM((96,), jnp.int32),
            pltpu.VMEM((96,), jnp.int32),
            pltpu.VMEM((64,), jnp.int32),
            pltpu.VMEM((64,), jnp.float32),
            pltpu.VMEM((80,), jnp.float32),
            pltpu.VMEM((16,), jnp.int32),
            pltpu.VMEM((16,), jnp.int32),
            pltpu.VMEM((16, E), jnp.float32),
            pltpu.SemaphoreType.DMA,
        ],
    )
    return f(lg_pad, gx1, cur, gum, emb)


def _bias_body(ce_ref, w_ref, out_ref):
    ce = ce_ref[...]
    w = w_ref[...]
    t1 = jnp.sum(w * w, axis=1)
    t3 = jnp.sum(ce * ce, axis=1)
    t2 = lax.dot_general(
        ce.astype(jnp.bfloat16), w.astype(jnp.bfloat16),
        (((1,), (1,)), ((), ())), preferred_element_type=jnp.float32)
    out_ref[...] = 2.0 * t2 - t1[None, :] - t3[:, None]


def _bias_matmul(ce, embed_weight):
    R = ce.shape[0]
    nv = pl.cdiv(V, _VB)
    return pl.pallas_call(
        _bias_body,
        grid=(nv,),
        in_specs=[
            pl.BlockSpec((R, E), lambda j: (0, 0)),
            pl.BlockSpec((_VB, E), lambda j: (j, 0)),
        ],
        out_specs=pl.BlockSpec((R, _VB), lambda j: (0, j)),
        out_shape=jax.ShapeDtypeStruct((R, V), jnp.float32),
    )(ce, embed_weight)


def kernel(gx, logits, embed_weight, cur_token_ids):
    B, L, _ = gx.shape
    lg_pad = jnp.pad(
        logits[:, PROMPT_LENGTH:, :].reshape(B * L, V),
        ((0, 0), (0, VPAD - V)), constant_values=NEG_INF)
    gx1 = gx.reshape(-1)
    cur = cur_token_ids[:, PROMPT_LENGTH:].reshape(-1)
    gum = jax.random.gumbel(jax.random.key(42), (B, L, K_VAL), dtype=jnp.float32)
    gum = jnp.pad(gum.reshape(B * L, K_VAL), ((0, 0), (0, 80 - K_VAL)))

    ce, _ids = _sc_sample(lg_pad, gx1, cur, gum, embed_weight)
    bias = _bias_matmul(ce, embed_weight)
    return bias.reshape(B, L, V)


# speedup vs baseline: 11.8085x; 1.0002x over previous
import jax
import jax.numpy as jnp
from jax import lax
from jax.experimental import pallas as pl
from jax.experimental.pallas import tpu as pltpu
from jax.experimental.pallas import tpu_sc as plsc

EPS = 1e-10
PROMPT_LENGTH = 32
K_VAL = 50

V = 50257
E = 768
VPAD = 50272
NVEC = VPAD // 16
NEG_INF = float("-inf")
IMIN = -2147483648

_VB = 1024

_NC = 2
_NS = 16
_NW = _NC * _NS


def _splat(x):
    return jnp.full((16,), x, jnp.int32)


def _hist_walk(hist, slab_base, krem):
    lanes = lax.iota(jnp.int32, 16)
    zero = _splat(0)

    def chunk(c2, carry):
        carry_cnt, done, dstar, cntab = carry
        c = 15 - c2
        didx = jnp.left_shift(_splat(c * 16) + lanes, 4)
        t = zero
        for l in range(16):
            t = t + plsc.load_gather(hist, [didx + _splat(slab_base + l)])
        r = lax.rev(t, (0,))
        cs = plsc.cumsum(r) + carry_cnt
        m = cs >= krem
        found = jnp.any(m)
        j0 = plsc.all_reduce_ffs(m)
        dstar_new = _splat(c * 16 + 15) - j0
        below = jnp.where(lanes < j0, r, zero)
        cntab_new = carry_cnt + _splat(jnp.sum(below))
        upd = jnp.logical_and(found, jnp.logical_not(done))
        dstar = jnp.where(upd, dstar_new, dstar)
        cntab = jnp.where(upd, cntab_new, cntab)
        done = jnp.logical_or(done, found)
        carry_cnt = carry_cnt + _splat(jnp.sum(t))
        return carry_cnt, done, dstar, cntab

    init = (zero, jnp.zeros((16,), jnp.bool_), zero, zero)
    _, _, dstar, cntab = lax.fori_loop(0, 16, chunk, init)
    return dstar, cntab


def _sc_body(lg, gx1, cur, gum_h, emb, ce_out, ids_out,
             row_v, hist, cand_v, cand_i, gflat, gxv, gum_v, cur_v,
             ids_v, ce_v, sem):
    wid = lax.axis_index("s") * _NC + lax.axis_index("c")
    base = wid * 16
    lanes = lax.iota(jnp.int32, 16)
    ones = _splat(1)
    zero = _splat(0)
    izeros = jnp.zeros((16,), jnp.int32)
    fzeros = jnp.zeros((16,), jnp.float32)

    pltpu.sync_copy(cur.at[pl.ds(base, 16)], cur_v)

    def row_step(i, aids):
        r = base + i
        pltpu.sync_copy(lg.at[r], row_v)
        pltpu.sync_copy(gum_h.at[r], gum_v)

        @plsc.parallel_loop(0, 1024, unroll=8)
        def _zr(j):
            hist[pl.ds(j * 16, 16)] = izeros
        for a in range(4):
            cand_v[pl.ds(a * 16, 16)] = izeros
            cand_i[pl.ds(a * 16, 16)] = izeros

        @plsc.parallel_loop(0, NVEC, unroll=4)
        def _p0(j):
            x = row_v[pl.ds(j * 16, 16)]
            b = lax.bitcast_convert_type(x, jnp.int32)
            s = jnp.right_shift(b, 31)
            v = jnp.bitwise_xor(b, jnp.bitwise_and(s, 0x7FFFFFFF))
            row_v[pl.ds(j * 16, 16)] = lax.bitcast_convert_type(v, jnp.float32)
            d1 = jnp.right_shift(v, 24) + 128
            idx = jnp.left_shift(d1, 4) + lanes
            plsc.addupdate_scatter(hist, [idx], ones)

        krem = _splat(K_VAL)
        b1, cntab = _hist_walk(hist, 0, krem)
        krem = krem - cntab
        p1 = b1 - 128

        @plsc.parallel_loop(0, NVEC, unroll=4)
        def _p2s(j):
            v = lax.bitcast_convert_type(row_v[pl.ds(j * 16, 16)], jnp.int32)
            pm = jnp.right_shift(v, 24) == p1
            d = jnp.bitwise_and(jnp.right_shift(v, 16), 0xFF)
            idx = 4096 + jnp.left_shift(d, 4) + lanes
            plsc.addupdate_scatter(hist, [idx], ones, mask=pm)
        b2, cntab = _hist_walk(hist, 4096, krem)
        krem = krem - cntab
        p2 = p1 * 256 + b2

        @plsc.parallel_loop(0, NVEC, unroll=4)
        def _p3s(j):
            v = lax.bitcast_convert_type(row_v[pl.ds(j * 16, 16)], jnp.int32)
            pm = jnp.right_shift(v, 16) == p2
            d = jnp.bitwise_and(jnp.right_shift(v, 8), 0xFF)
            idx = 8192 + jnp.left_shift(d, 4) + lanes
            plsc.addupdate_scatter(hist, [idx], ones, mask=pm)
        b3, cntab = _hist_walk(hist, 8192, krem)
        krem = krem - cntab
        p3 = p2 * 256 + b3

        @plsc.parallel_loop(0, NVEC, unroll=4)
        def _p4s(j):
            v = lax.bitcast_convert_type(row_v[pl.ds(j * 16, 16)], jnp.int32)
            pm = jnp.right_shift(v, 8) == p3
            d = jnp.bitwise_and(v, 0xFF)
            idx = 12288 + jnp.left_shift(d, 4) + lanes
            plsc.addupdate_scatter(hist, [idx], ones, mask=pm)
        b4, cntab = _hist_walk(hist, 12288, krem)
        quota = krem - cntab
        tval = p3 * 256 + b4

        @plsc.parallel_loop(0, NVEC, unroll=2, carry=(zero, zero))
        def coll(j, carry):
            ncoll, eqcnt = carry
            v = lax.bitcast_convert_type(row_v[pl.ds(j * 16, 16)], jnp.int32)
            gt = v > tval
            eq = v == tval
            pref = plsc.cumsum(eq.astype(jnp.int32)) + eqcnt
            acc = jnp.logical_and(eq, pref <= quota)
            msk = jnp.logical_or(gt, acc)
            pos = plsc.cumsum(msk.astype(jnp.int32)) - 1 + ncoll
            pos = jnp.where(msk, pos, _splat(64) + lanes)
            gidx = _splat(j * 16) + lanes
            plsc.addupdate_scatter(cand_v, [pos], v, mask=msk)
            plsc.addupdate_scatter(cand_i, [pos], gidx, mask=msk)
            ncoll = ncoll + _splat(jnp.sum(msk.astype(jnp.int32)))
            eqcnt = eqcnt + _splat(jnp.sum(eq.astype(jnp.int32)))
            return ncoll, eqcnt
        ncoll_f, _eq_f = coll

        rv = r * V
        tail = cand_v[pl.ds(48, 16)]
        cand_v[pl.ds(48, 16)] = jnp.where(lanes < 2, tail, _splat(IMIN))
        cu = []
        ci = []
        for a in range(4):
            cu.append(cand_v[pl.ds(a * 16, 16)])
            ci.append(cand_i[pl.ds(a * 16, 16)])
            safe = jnp.clip(ci[a], 0, V - 1)
            gflat[pl.ds(a * 16, 16)] = safe + _splat(rv)
        pltpu.async_copy(gx1.at[gflat], gxv, sem).wait()

        @plsc.parallel_loop(0, 64, unroll=4,
                            carry=(izeros, izeros, izeros, izeros))
        def rank_step(t, rk):
            ts = _splat(t)
            ub = plsc.load_gather(cand_v, [ts])
            ib = plsc.load_gather(cand_i, [ts])
            out = []
            for a in range(4):
                c = jnp.logical_or(
                    ub > cu[a],
                    jnp.logical_and(ub == cu[a], ib < ci[a]))
                out.append(rk[a] + c.astype(jnp.int32))
            return tuple(out)
        ranks = rank_step

        curid = plsc.load_gather(cur_v, [_splat(i)])
        best = jnp.full((16,), NEG_INF, jnp.float32)
        scs = []
        for a in range(4):
            gv = plsc.load_gather(gum_v, [ranks[a]])
            xg = gxv[pl.ds(a * 16, 16)]
            neg = -xg
            dist = jnp.where(ci[a] == curid, neg * EPS, neg)
            sc = dist + gv
            if a == 3:
                sc = jnp.where(lanes < 2, sc, jnp.full((16,), NEG_INF, jnp.float32))
            scs.append(sc)
            best = jnp.maximum(best, sc)
        msp = jnp.full((16,), jnp.max(best), jnp.float32)
        rmin = _splat(64)
        for a in range(4):
            rmin = jnp.minimum(rmin, jnp.where(scs[a] == msp, ranks[a], _splat(64)))
        rmin = _splat(jnp.min(rmin))
        aid = zero
        for a in range(4):
            aid = aid + jnp.where(ranks[a] == rmin, ci[a], zero)
        aid = _splat(jnp.sum(aid))
        return jnp.where(lanes == _splat(i), aid, aids)

    aids = lax.fori_loop(0, 16, row_step, izeros)
    ids_v[...] = jnp.clip(aids, 0, V - 1)
    pltpu.async_copy(emb.at[ids_v], ce_v, sem).wait()
    pltpu.sync_copy(ce_v, ce_out.at[pl.ds(base, 16)])
    pltpu.sync_copy(ids_v, ids_out.at[pl.ds(base, 16)])


@jax.jit
def _sc_sample(lg_pad, gx1, cur, gum, emb):
    mesh = plsc.VectorSubcoreMesh(core_axis_name="c", subcore_axis_name="s")
    f = pl.kernel(
        _sc_body,
        out_type=[
            jax.ShapeDtypeStruct((_NW * 16, E), jnp.float32),
            jax.ShapeDtypeStruct((_NW * 16,), jnp.int32),
        ],
        mesh=mesh,
        compiler_params=pltpu.CompilerParams(needs_layout_passes=False),
        scratch_types=[
            pltpu.VMEM((VPAD,), jnp.float32),
            pltpu.VMEM((16384,), jnp.int32),
            pltpu.VMEM((96,), jnp.int32),
            pltpu.VMEM((96,), jnp.int32),
            pltpu.VMEM((64,), jnp.int32),
            pltpu.VMEM((64,), jnp.float32),
            pltpu.VMEM((80,), jnp.float32),
            pltpu.VMEM((16,), jnp.int32),
            pltpu.VMEM((16,), jnp.int32),
            pltpu.VMEM((16, E), jnp.float32),
            pltpu.SemaphoreType.DMA,
        ],
    )
    return f(lg_pad, gx1, cur, gum, emb)


def _bias_body(ce_ref, w_ref, out_ref):
    ce = ce_ref[...]
    w = w_ref[...]
    t1 = jnp.sum(w * w, axis=1)
    t3 = jnp.sum(ce * ce, axis=1)
    t2 = lax.dot_general(
        ce.astype(jnp.bfloat16), w.astype(jnp.bfloat16),
        (((1,), (1,)), ((), ())), preferred_element_type=jnp.float32)
    out_ref[...] = 2.0 * t2 - t1[None, :] - t3[:, None]


def _bias_matmul(ce, embed_weight):
    R = ce.shape[0]
    nv = pl.cdiv(V, _VB)
    return pl.pallas_call(
        _bias_body,
        grid=(nv,),
        in_specs=[
            pl.BlockSpec((R, E), lambda j: (0, 0)),
            pl.BlockSpec((_VB, E), lambda j: (j, 0)),
        ],
        out_specs=pl.BlockSpec((R, _VB), lambda j: (0, j)),
        out_shape=jax.ShapeDtypeStruct((R, V), jnp.float32),
    )(ce, embed_weight)


def kernel(gx, logits, embed_weight, cur_token_ids):
    B, L, _ = gx.shape
    lg_pad = jnp.pad(
        logits[:, PROMPT_LENGTH:, :].reshape(B * L, V),
        ((0, 0), (0, VPAD - V)), constant_values=NEG_INF)
    gx1 = gx.reshape(-1)
    cur = cur_token_ids[:, PROMPT_LENGTH:].reshape(-1)
    gum = jax.random.gumbel(jax.random.key(42), (B, L, K_VAL), dtype=jnp.float32)
    gum = jnp.pad(gum.reshape(B * L, K_VAL), ((0, 0), (0, 80 - K_VAL)))

    ce, _ids = _sc_sample(lg_pad, gx1, cur, gum, embed_weight)
    bias = _bias_matmul(ce, embed_weight)
    return bias.reshape(B, L, V)
